# baseline (speedup 1.0000x reference)
import jax
import jax.numpy as jnp
from jax.experimental import pallas as pl
from jax.experimental.pallas import tpu as pltpu

_EPS = 1e-6
_BM = 256
_BW = 64


def _body(x_ref, w_hbm, a_ref, b_ref, scale_ref, lnb_ref, bias_ref,
          o_ref, w_sc, rb_sc, y_sc, w_buf, w_sem):
    i = pl.program_id(0)
    H = x_ref.shape[1]
    n_chunks = H // _BW

    @pl.when(i == 0)
    def _fold():
        def chunk_copy(k, slot):
            return pltpu.make_async_copy(
                w_hbm.at[pl.ds(k * _BW, _BW), :], w_buf.at[slot],
                w_sem.at[slot])

        chunk_copy(0, 0).start()
        rb = bias_ref[...]
        for k in range(n_chunks):
            if k + 1 < n_chunks:
                chunk_copy(k + 1, (k + 1) % 2).start()
            chunk_copy(k, k % 2).wait()
            rows = pl.ds(k * _BW, _BW)
            ab = jax.lax.dot_general(
                a_ref[:, rows], b_ref[...],
                dimension_numbers=(((0,), (0,)), ((), ())),
                preferred_element_type=jnp.float32)
            wp = w_buf[k % 2] + ab
            g, h = k // 2, (k % 2) * _BW
            scol = jnp.transpose(scale_ref[g:g + 1, h:h + _BW])
            lcol = jnp.transpose(lnb_ref[g:g + 1, h:h + _BW])
            w_sc[rows, :] = (wp * scol).astype(jnp.bfloat16)
            rb = rb + jnp.sum(wp * lcol, axis=0, keepdims=True)
        rb_sc[...] = rb

    cur = jax.lax.rem(i, 2)
    prev = jax.lax.rem(i + 1, 2)

    acc = jnp.dot(y_sc[prev], w_sc[...], preferred_element_type=jnp.float32)
    o_ref[...] = acc + rb_sc[...]

    xv = x_ref[...]
    inv_h = 1.0 / H
    s1 = jnp.sum(xv, axis=1, keepdims=True)
    s2 = jnp.sum(xv * xv, axis=1, keepdims=True)
    mean = s1 * inv_h
    var = s2 * inv_h - mean * mean
    rstd = jax.lax.rsqrt(var + _EPS)
    y_sc[cur] = ((xv - mean) * rstd).astype(jnp.bfloat16)


def kernel(x, scale, ln_bias, kernel, lora_a, lora_b, bias):
    B, S, H = x.shape
    F = kernel.shape[1]
    M = B * S
    n_tiles = M // _BM
    x2 = x.reshape(M, H)

    out = pl.pallas_call(
        _body,
        out_shape=jax.ShapeDtypeStruct((M, F), jnp.float32),
        grid=(n_tiles + 1,),
        in_specs=[
            pl.BlockSpec((_BM, H), lambda i: (jnp.minimum(i, n_tiles - 1), 0)),
            pl.BlockSpec(memory_space=pl.ANY),
            pl.BlockSpec(memory_space=pltpu.VMEM),
            pl.BlockSpec(memory_space=pltpu.VMEM),
            pl.BlockSpec(memory_space=pltpu.VMEM),
            pl.BlockSpec(memory_space=pltpu.VMEM),
            pl.BlockSpec(memory_space=pltpu.VMEM),
        ],
        out_specs=pl.BlockSpec((_BM, F),
                               lambda i: (jnp.maximum(i - 1, 0), 0)),
        scratch_shapes=[
            pltpu.VMEM((H, F), jnp.bfloat16),
            pltpu.VMEM((1, F), jnp.float32),
            pltpu.VMEM((2, _BM, H), jnp.bfloat16),
            pltpu.VMEM((2, _BW, F), jnp.float32),
            pltpu.SemaphoreType.DMA((2,)),
        ],
        compiler_params=pltpu.CompilerParams(
            dimension_semantics=("arbitrary",),
            vmem_limit_bytes=60 * 1024 * 1024,
        ),
        name="ln_dense_fold",
    )(x2, kernel, lora_a.T, lora_b, scale.reshape(H // 128, 128),
      ln_bias.reshape(H // 128, 128), bias.reshape(1, F))
    return out.reshape(B, S, F)


# speedup vs baseline: 1.0656x; 1.0656x over previous
import jax
import jax.numpy as jnp
from jax.experimental import pallas as pl
from jax.experimental.pallas import tpu as pltpu

_EPS = 1e-6
_BM = 256
_BW = 64


def _body(x_ref, w_hbm, a_ref, b_ref, scale_ref, lnb_ref, bias_ref,
          o_ref, w_sc, rb_sc, y_sc, w_buf, w_sem):
    i = pl.program_id(0)
    H = x_ref.shape[1]
    n_chunks = H // _BW

    @pl.when(i == 0)
    def _fold():
        def chunk_copy(k, slot):
            return pltpu.make_async_copy(
                w_hbm.at[pl.ds(k * _BW, _BW), :], w_buf.at[slot],
                w_sem.at[slot])

        for k0 in range(3):
            chunk_copy(k0, k0 % 4).start()
        rb = bias_ref[...]
        for k in range(n_chunks):
            if k + 3 < n_chunks:
                chunk_copy(k + 3, (k + 3) % 4).start()
            chunk_copy(k, k % 4).wait()
            rows = pl.ds(k * _BW, _BW)
            ab = jax.lax.dot_general(
                a_ref[:, rows], b_ref[...],
                dimension_numbers=(((0,), (0,)), ((), ())),
                preferred_element_type=jnp.float32)
            wp = w_buf[k % 4] + ab
            g, h = k // 2, (k % 2) * _BW
            scol = jnp.transpose(scale_ref[g:g + 1, h:h + _BW])
            lcol = jnp.transpose(lnb_ref[g:g + 1, h:h + _BW])
            w_sc[rows, :] = (wp * scol).astype(jnp.bfloat16)
            rb = rb + jnp.sum(wp * lcol, axis=0, keepdims=True)
        rb_sc[...] = rb

    cur = jax.lax.rem(i, 2)
    prev = jax.lax.rem(i + 1, 2)

    acc = jnp.dot(y_sc[prev], w_sc[...], preferred_element_type=jnp.float32)
    o_ref[...] = acc + rb_sc[...]

    xv = x_ref[...]
    inv_h = 1.0 / H
    s1 = jnp.sum(xv, axis=1, keepdims=True)
    s2 = jnp.sum(xv * xv, axis=1, keepdims=True)
    mean = s1 * inv_h
    var = s2 * inv_h - mean * mean
    rstd = jax.lax.rsqrt(var + _EPS)
    y_sc[cur] = ((xv - mean) * rstd).astype(jnp.bfloat16)


def kernel(x, scale, ln_bias, kernel, lora_a, lora_b, bias):
    B, S, H = x.shape
    F = kernel.shape[1]
    M = B * S
    n_tiles = M // _BM
    x2 = x.reshape(M, H)

    out = pl.pallas_call(
        _body,
        out_shape=jax.ShapeDtypeStruct((M, F), jnp.float32),
        grid=(n_tiles + 1,),
        in_specs=[
            pl.BlockSpec((_BM, H), lambda i: (jnp.minimum(i, n_tiles - 1), 0)),
            pl.BlockSpec(memory_space=pl.ANY),
            pl.BlockSpec(memory_space=pltpu.VMEM),
            pl.BlockSpec(memory_space=pltpu.VMEM),
            pl.BlockSpec(memory_space=pltpu.VMEM),
            pl.BlockSpec(memory_space=pltpu.VMEM),
            pl.BlockSpec(memory_space=pltpu.VMEM),
        ],
        out_specs=pl.BlockSpec((_BM, F),
                               lambda i: (jnp.maximum(i - 1, 0), 0)),
        scratch_shapes=[
            pltpu.VMEM((H, F), jnp.bfloat16),
            pltpu.VMEM((1, F), jnp.float32),
            pltpu.VMEM((2, _BM, H), jnp.bfloat16),
            pltpu.VMEM((4, _BW, F), jnp.float32),
            pltpu.SemaphoreType.DMA((4,)),
        ],
        compiler_params=pltpu.CompilerParams(
            dimension_semantics=("arbitrary",),
            vmem_limit_bytes=62 * 1024 * 1024,
        ),
        name="ln_dense_fold",
    )(x2, kernel, lora_a.T, lora_b, scale.reshape(H // 128, 128),
      ln_bias.reshape(H // 128, 128), bias.reshape(1, F))
    return out.reshape(B, S, F)
